# Initial kernel scaffold; baseline (speedup 1.0000x reference)
import functools

import jax
import jax.numpy as jnp
from jax import lax
from jax.experimental import pallas as pl
from jax.experimental.pallas import tpu as pltpu
from jax.experimental.pallas import tpu_sc as plsc

N = 10000
D = 128
E = 320000
NPAD = 10240
NC = 2
NS = 16
NW = NC * NS
CH = 128
EPT = NPAD
EPAD = EPT * NW
NCHUNK = EPT // CH
NB = 5
NG = NCHUNK // NB
RPT = NPAD // NS
RB = 1024

_mesh = plsc.VectorSubcoreMesh(core_axis_name="c", subcore_axis_name="s")


@functools.partial(
    pl.kernel,
    out_type=jax.ShapeDtypeStruct((NC, 8, NPAD), jnp.float32),
    mesh=_mesh,
    scratch_types=[
        pltpu.VMEM((NCHUNK, CH), jnp.int32),
        pltpu.VMEM((CH,), jnp.float32),
        pltpu.VMEM((6, RPT), jnp.float32),
        pltpu.VMEM_SHARED((6, NPAD), jnp.float32),
        pltpu.SemaphoreType.DMA,
    ],
)
def _deg_kernel(idx6, out, idx_v, ones_v, zb_v, deg_sh, sem_s):
    c = lax.axis_index("c")
    s = lax.axis_index("s")
    w = c * NS + s
    for k in range(CH // 16):
        ones_v[pl.ds(k * 16, 16)] = jnp.ones((16,), jnp.float32)
    for h in range(6):
        def _z(k, carry, h=h):
            zb_v[h, pl.ds(k * 16, 16)] = jnp.zeros((16,), jnp.float32)
            return carry
        lax.fori_loop(0, RPT // 16, _z, 0)
    pltpu.sync_copy(zb_v, deg_sh.at[:, pl.ds(s * RPT, RPT)])
    plsc.subcore_barrier()

    PIPE = 8
    for h in range(6):
        pltpu.sync_copy(idx6.at[h, w], idx_v)

        def _body(j, carry, h=h):
            pltpu.async_copy(ones_v, deg_sh.at[h].at[idx_v.at[j]], sem_s,
                             add=True)
            @pl.when(j >= PIPE)
            def _w():
                pltpu.make_async_copy(
                    ones_v, deg_sh.at[h].at[idx_v.at[0]], sem_s).wait()
            return carry

        lax.fori_loop(0, NCHUNK, _body, 0)
        for _ in range(PIPE):
            pltpu.make_async_copy(
                ones_v, deg_sh.at[h].at[idx_v.at[0]], sem_s).wait()

    plsc.subcore_barrier()
    for h in range(6):
        pltpu.sync_copy(deg_sh.at[h, pl.ds(s * RPT, RPT)],
                        out.at[c, h, pl.ds(s * RPT, RPT)])


def _make_agg_kernel(t):
    hs, hd = 2 * t, 2 * t + 1

    @functools.partial(
        pl.kernel,
        out_type=jax.ShapeDtypeStruct((NC, NPAD, D), jnp.float32),
        mesh=_mesh,
        scratch_types=(
            [pltpu.VMEM((NCHUNK, CH), jnp.int32),
             pltpu.VMEM((NCHUNK, CH), jnp.int32),
             pltpu.VMEM((NB, CH, D), jnp.float32),
             pltpu.VMEM_SHARED((NPAD, D), jnp.float32)]
            + [pltpu.SemaphoreType.DMA] * (2 * NB)
        ),
    )
    def _agg(idx6, xs, out, si_v, di_v, rows_v, agg_sh, *sems):
        c = lax.axis_index("c")
        s = lax.axis_index("s")
        w = c * NS + s
        gs = sems[:NB]
        ss = sems[NB:]

        def _z(i, carry):
            for k in range(D // 16):
                rows_v[0, i, pl.ds(k * 16, 16)] = jnp.zeros((16,), jnp.float32)
            return carry
        lax.fori_loop(0, CH, _z, 0)
        for r in range(RPT // CH):
            pltpu.sync_copy(rows_v.at[0],
                            agg_sh.at[pl.ds(s * RPT + r * CH, CH)])
        plsc.subcore_barrier()

        pltpu.sync_copy(idx6.at[hs, w], si_v)
        pltpu.sync_copy(idx6.at[hd, w], di_v)

        def fire_g(j, b):
            pltpu.async_copy(xs.at[si_v.at[j]], rows_v.at[b], gs[b])

        def wait_g(b):
            pltpu.make_async_copy(xs.at[si_v.at[0]], rows_v.at[b],
                                  gs[b]).wait()

        def fire_s(j, b):
            pltpu.async_copy(rows_v.at[b], agg_sh.at[di_v.at[j]], ss[b],
                             add=True)

        def wait_s(b):
            pltpu.make_async_copy(rows_v.at[b], agg_sh.at[di_v.at[0]],
                                  ss[b]).wait()

        for b in range(NB):
            fire_g(b, b)

        def _group(g, carry):
            base = g * NB
            for b in range(NB):
                wait_g(b)
                fire_s(base + b, b)
            for b in range(NB):
                @pl.when(g < NG - 1)
                def _p(b=b, base=base):
                    wait_s(b)
                    fire_g(base + NB + b, b)
            return carry

        lax.fori_loop(0, NG, _group, 0)
        for b in range(NB):
            wait_s(b)
        plsc.subcore_barrier()
        pltpu.sync_copy(agg_sh.at[pl.ds(s * RPT, RPT)],
                        out.at[c, pl.ds(s * RPT, RPT)])

    return _agg


_agg_kernels = [_make_agg_kernel(t) for t in range(3)]


def _norms_xs0_body(dref, f0_ref, norms_ref, xs0_ref):
    d = dref[0] + dref[1]
    n6 = lax.rsqrt(jnp.maximum(d[0:6], 1.0))
    y8 = jnp.concatenate(
        [n6, jnp.zeros((2, n6.shape[1]), jnp.float32)], axis=0)
    i8 = jnp.eye(8, dtype=jnp.float32)
    nt = lax.dot_general(y8, i8, (((0,), (0,)), ((), ())),
                         preferred_element_type=jnp.float32)
    norms_ref[...] = nt
    xs0_ref[...] = f0_ref[...] * nt[:, 0:1]


def _norms_xs0(deg, f0p):
    return pl.pallas_call(
        _norms_xs0_body,
        grid=(NPAD // RB,),
        in_specs=[
            pl.BlockSpec((2, 8, RB), lambda i: (0, 0, i)),
            pl.BlockSpec((RB, D), lambda i: (i, 0)),
        ],
        out_specs=[
            pl.BlockSpec((RB, 8), lambda i: (i, 0)),
            pl.BlockSpec((RB, D), lambda i: (i, 0)),
        ],
        out_shape=[
            jax.ShapeDtypeStruct((NPAD, 8), jnp.float32),
            jax.ShapeDtypeStruct((NPAD, D), jnp.float32),
        ],
    )(deg, f0p)


def _make_m_body(t):
    def body(*refs):
        if t == 0:
            pref, nref, wref, bref, oref, xref = refs
        elif t == 1:
            pref, nref, wref, bref, vref, oref, xref, dref = refs
        else:
            pref, nref, wref, bref, vref, oref, dref = refs
        nt = nref[...]
        gblk = (pref[0] + pref[1]) * nt[:, 2 * t + 1:2 * t + 2]
        o = jnp.dot(gblk, wref[...],
                    preferred_element_type=jnp.float32) + bref[0:1, :]
        oref[...] = o
        if t < 2:
            xref[...] = o * nt[:, 2 * t + 2:2 * t + 3]
        if t > 0:
            dref[...] = o - vref[...]
    return body


def _matmul_step(t, part, norms, W, b8, prev):
    in_specs = [
        pl.BlockSpec((2, RB, D), lambda i: (0, i, 0)),
        pl.BlockSpec((RB, 8), lambda i: (i, 0)),
        pl.BlockSpec((D, D), lambda i: (0, 0)),
        pl.BlockSpec((8, D), lambda i: (0, 0)),
    ]
    args = [part, norms, W, b8]
    if t > 0:
        in_specs.append(pl.BlockSpec((RB, D), lambda i: (i, 0)))
        args.append(prev)
    n_out = 2 if t != 1 else 3
    out_specs = [pl.BlockSpec((RB, D), lambda i: (i, 0))] * n_out
    out_shape = [jax.ShapeDtypeStruct((NPAD, D), jnp.float32)] * n_out
    return pl.pallas_call(
        _make_m_body(t),
        grid=(NPAD // RB,),
        in_specs=in_specs,
        out_specs=out_specs,
        out_shape=out_shape,
    )(*args)


def kernel(features_0, features_1, features_2, edge_index_0, edge_index_1,
           edge_index_2, W_0, W_1, W_2, b_0, b_1, b_2):
    del features_1, features_2

    cols = []
    for e in (edge_index_0, edge_index_1, edge_index_2):
        for r in range(2):
            cols.append(
                jnp.full((EPAD,), N, jnp.int32).at[:E].set(e[r])
                .reshape(NW, NCHUNK, CH))
    idx6 = jnp.stack(cols)

    f0p = jnp.zeros((NPAD, D), jnp.float32).at[:N].set(features_0)
    b8s = [jnp.broadcast_to(b.reshape(1, D), (8, D)) for b in (b_0, b_1, b_2)]
    Ws = (W_0, W_1, W_2)

    deg = _deg_kernel(idx6)
    norms, xs0 = _norms_xs0(deg, f0p)

    p0 = _agg_kernels[0](idx6, xs0)
    out0, xs1 = _matmul_step(0, p0, norms, Ws[0], b8s[0], None)
    p1 = _agg_kernels[1](idx6, xs1)
    out1, xs2, diff1 = _matmul_step(1, p1, norms, Ws[1], b8s[1], out0)
    p2 = _agg_kernels[2](idx6, xs2)
    out2, diff2 = _matmul_step(2, p2, norms, Ws[2], b8s[2], out1)

    o0, o1, o2 = out0[:N], out1[:N], out2[:N]
    return (o0, o1, o2, o0, diff1[:N], diff2[:N])


# speedup vs baseline: 3.0825x; 3.0825x over previous
import functools

import jax
import jax.numpy as jnp
from jax import lax
from jax.experimental import pallas as pl
from jax.experimental.pallas import tpu as pltpu
from jax.experimental.pallas import tpu_sc as plsc

N = 10000
D = 128
E = 320000
NPAD = 10240
NC = 2
NS = 16
NW = NC * NS
CH = 64
EPT = NPAD
EPAD = EPT * NW
NCHUNK = EPT // CH
NB = 2
NG = NCHUNK // NB
RPT = NPAD // NS
RB = 1024

_mesh = plsc.VectorSubcoreMesh(core_axis_name="c", subcore_axis_name="s")


NROW = NPAD // 128


@functools.partial(
    pl.kernel,
    out_type=jax.ShapeDtypeStruct((NW, 6, NROW, 128), jnp.float32),
    mesh=_mesh,
    scratch_types=[
        pltpu.VMEM((NCHUNK // 5, CH), jnp.int32),
        pltpu.VMEM((NPAD,), jnp.float32),
        pltpu.VMEM((NROW, 128), jnp.float32),
        pltpu.VMEM((1, NROW), jnp.int32),
        pltpu.SemaphoreType.DMA,
    ],
    compiler_params=pltpu.CompilerParams(needs_layout_passes=False),
)
def _deg_kernel(i0, i1, i2, i3, i4, i5, out, idx_v, hist_v, hist2_v,
                ii_v, sem):
    c = lax.axis_index("c")
    s = lax.axis_index("s")
    w = c * NS + s

    def _fill_ii(m, carry):
        ii_v[0, pl.ds(m * 16, 16)] = lax.iota(jnp.int32, 16) + m * 16
        return carry
    lax.fori_loop(0, NROW // 16, _fill_ii, 0)

    ones = jnp.ones((16,), jnp.float32)
    NSEC = 5
    SEC = NCHUNK // NSEC
    for h, href in enumerate((i0, i1, i2, i3, i4, i5)):
        def _zero(i, carry):
            hist_v[pl.ds(i * 16, 16)] = jnp.zeros((16,), jnp.float32)
            return carry
        lax.fori_loop(0, NPAD // 16, _zero, 0)

        for sec in range(NSEC):
            pltpu.sync_copy(href.at[w, pl.ds(sec * SEC, SEC)], idx_v)

            def _acc(j, carry):
                for k in range(CH // 16):
                    v = idx_v[j, pl.ds(k * 16, 16)]
                    plsc.addupdate_scatter(hist_v, [v], ones)
                return carry
            lax.fori_loop(0, SEC, _acc, 0)

        def _pack(i, carry):
            for k in range(128 // 16):
                hist2_v[i, pl.ds(k * 16, 16)] = (
                    hist_v[pl.ds(i * 128 + k * 16, 16)])
            return carry
        lax.fori_loop(0, NROW, _pack, 0)
        pltpu.async_copy(hist2_v, out.at[w, h].at[ii_v.at[0]], sem).wait()


@functools.partial(
    pl.kernel,
    out_type=jax.ShapeDtypeStruct((NC, NPAD, D), jnp.float32),
    mesh=_mesh,
    scratch_types=(
        [pltpu.VMEM((2, NB, CH), jnp.int32),
         pltpu.VMEM((2, NB, CH), jnp.int32),
         pltpu.VMEM((NB, CH, D), jnp.float32),
         pltpu.VMEM((RPT // CH, CH), jnp.int32),
         pltpu.VMEM_SHARED((NPAD, D), jnp.float32)]
        + [pltpu.SemaphoreType.DMA] * (2 * NB + 2)
    ),
)
def _agg_kernel(srcC, dstC, xs, out, si_v, di_v, rows_v, ii_v, agg_sh,
                *sems):
    c = lax.axis_index("c")
    s = lax.axis_index("s")
    w = c * NS + s
    gs = sems[:NB]
    ss = sems[NB:2 * NB]
    isem = sems[2 * NB:]
    NZ = RPT // CH

    def _z(i, carry):
        for k in range(D // 16):
            rows_v[0, i, pl.ds(k * 16, 16)] = jnp.zeros((16,), jnp.float32)
        return carry
    lax.fori_loop(0, CH, _z, 0)
    for r in range(NZ):
        def _fill_ii(m, carry, r=r):
            ii_v[r, pl.ds(m * 16, 16)] = (lax.iota(jnp.int32, 16)
                                          + s * RPT + r * CH + m * 16)
            return carry
        lax.fori_loop(0, CH // 16, _fill_ii, 0)
    for r in range(NZ):
        pltpu.async_copy(rows_v.at[0], agg_sh.at[ii_v.at[r]], isem[0])
    for r in range(NZ):
        pltpu.make_async_copy(rows_v.at[0], agg_sh.at[ii_v.at[0]],
                              isem[0]).wait()
    plsc.subcore_barrier()

    def load_idx(g, p):
        pltpu.async_copy(srcC.at[w, pl.ds(g * NB, NB)], si_v.at[p], isem[p])
        pltpu.async_copy(dstC.at[w, pl.ds(g * NB, NB)], di_v.at[p], isem[p])

    def wait_idx(p):
        pltpu.make_async_copy(srcC.at[0, pl.ds(0, NB)], si_v.at[p],
                              isem[p]).wait()
        pltpu.make_async_copy(dstC.at[0, pl.ds(0, NB)], di_v.at[p],
                              isem[p]).wait()

    def fire_g(p, b):
        pltpu.async_copy(xs.at[si_v.at[p, b]], rows_v.at[b], gs[b])

    def wait_g(b):
        pltpu.make_async_copy(xs.at[si_v.at[0, 0]], rows_v.at[b],
                              gs[b]).wait()

    def fire_s(p, b):
        pltpu.async_copy(rows_v.at[b], agg_sh.at[di_v.at[p, b]], ss[b],
                         add=True)

    def wait_s(b):
        pltpu.make_async_copy(rows_v.at[b], agg_sh.at[di_v.at[0, 0]],
                              ss[b]).wait()

    load_idx(0, 0)
    load_idx(1, 1)
    wait_idx(0)
    for b in range(NB):
        fire_g(0, b)

    def _sg(gg, carry):
        for p in range(2):
            g = 2 * gg + p
            for b in range(NB):
                wait_g(b)
                fire_s(p, b)

            @pl.when(g < NG - 1)
            def _wi(p=p):
                wait_idx(1 - p)
            for b in range(NB):
                @pl.when(g < NG - 1)
                def _nx(p=p, b=b):
                    wait_s(b)
                    fire_g(1 - p, b)

            @pl.when(g + 2 < NG)
            def _ld(g=g, p=p):
                load_idx(g + 2, p)
        return carry

    lax.fori_loop(0, NG // 2, _sg, 0)
    for b in range(NB):
        wait_s(b)
    plsc.subcore_barrier()

    def fire_o(r, b):
        pltpu.async_copy(agg_sh.at[ii_v.at[r]], rows_v.at[b], gs[b])

    def wait_o(b):
        pltpu.make_async_copy(agg_sh.at[ii_v.at[0]], rows_v.at[b],
                              gs[b]).wait()

    def fire_w(r, b):
        pltpu.async_copy(rows_v.at[b], out.at[c].at[ii_v.at[r]], ss[b])

    def wait_w(b):
        pltpu.make_async_copy(rows_v.at[b], out.at[c].at[ii_v.at[0]],
                              ss[b]).wait()

    fire_o(0, 0)
    for r in range(NZ):
        b = r % 2
        wait_o(b)
        if r >= 1:
            wait_w(1 - b)
        if r + 1 < NZ:
            fire_o(r + 1, 1 - b)
        fire_w(r, b)
    wait_w((NZ - 1) % 2)


def _norms_xs0_body(dref, f0_ref, norms_ref, xs0_ref):
    d = jnp.sum(dref[...], axis=1)
    n6 = lax.rsqrt(jnp.maximum(d, 1.0))
    y8 = jnp.concatenate(
        [n6, jnp.zeros((2, n6.shape[1]), jnp.float32)], axis=0)
    i8 = jnp.eye(8, dtype=jnp.float32)
    nt = lax.dot_general(y8, i8, (((0,), (0,)), ((), ())),
                         preferred_element_type=jnp.float32)
    norms_ref[...] = nt
    xs0_ref[...] = f0_ref[...] * nt[:, 0:1]


def _norms_xs0(deg, f0p):
    return pl.pallas_call(
        _norms_xs0_body,
        grid=(NPAD // RB,),
        in_specs=[
            pl.BlockSpec((6, NW, RB), lambda i: (0, 0, i)),
            pl.BlockSpec((RB, D), lambda i: (i, 0)),
        ],
        out_specs=[
            pl.BlockSpec((RB, 8), lambda i: (i, 0)),
            pl.BlockSpec((RB, D), lambda i: (i, 0)),
        ],
        out_shape=[
            jax.ShapeDtypeStruct((NPAD, 8), jnp.float32),
            jax.ShapeDtypeStruct((NPAD, D), jnp.float32),
        ],
    )(deg, f0p)


def _make_m_body(t):
    def body(*refs):
        if t == 0:
            pref, nref, wref, bref, oref, xref = refs
        elif t == 1:
            pref, nref, wref, bref, vref, oref, xref, dref = refs
        else:
            pref, nref, wref, bref, vref, oref, dref = refs
        nt = nref[...]
        gblk = (pref[0] + pref[1]) * nt[:, 2 * t + 1:2 * t + 2]
        o = jnp.dot(gblk, wref[...],
                    preferred_element_type=jnp.float32) + bref[0:1, :]
        oref[...] = o
        if t < 2:
            xref[...] = o * nt[:, 2 * t + 2:2 * t + 3]
        if t > 0:
            dref[...] = o - vref[...]
    return body


def _matmul_step(t, part, norms, W, b8, prev):
    in_specs = [
        pl.BlockSpec((2, RB, D), lambda i: (0, i, 0)),
        pl.BlockSpec((RB, 8), lambda i: (i, 0)),
        pl.BlockSpec((D, D), lambda i: (0, 0)),
        pl.BlockSpec((8, D), lambda i: (0, 0)),
    ]
    args = [part, norms, W, b8]
    if t > 0:
        in_specs.append(pl.BlockSpec((RB, D), lambda i: (i, 0)))
        args.append(prev)
    n_out = 2 if t != 1 else 3
    out_specs = [pl.BlockSpec((RB, D), lambda i: (i, 0))] * n_out
    out_shape = [jax.ShapeDtypeStruct((NPAD, D), jnp.float32)] * n_out
    return pl.pallas_call(
        _make_m_body(t),
        grid=(NPAD // RB,),
        in_specs=in_specs,
        out_specs=out_specs,
        out_shape=out_shape,
    )(*args)


def kernel(features_0, features_1, features_2, edge_index_0, edge_index_1,
           edge_index_2, W_0, W_1, W_2, b_0, b_1, b_2):
    del features_1, features_2

    cols = []
    for e in (edge_index_0, edge_index_1, edge_index_2):
        for r in range(2):
            cols.append(
                jnp.full((EPAD,), N, jnp.int32).at[:E].set(e[r])
                .reshape(NW, NCHUNK, CH))

    f0p = jnp.zeros((NPAD, D), jnp.float32).at[:N].set(features_0)
    b8s = [jnp.broadcast_to(b.reshape(1, D), (8, D)) for b in (b_0, b_1, b_2)]
    Ws = (W_0, W_1, W_2)

    degw = _deg_kernel(*cols)
    deg = degw.reshape(NW, 6, NPAD).transpose(1, 0, 2)
    norms, xs0 = _norms_xs0(deg, f0p)

    p0 = _agg_kernel(cols[0], cols[1], xs0)
    out0, xs1 = _matmul_step(0, p0, norms, Ws[0], b8s[0], None)
    p1 = _agg_kernel(cols[2], cols[3], xs1)
    out1, xs2, diff1 = _matmul_step(1, p1, norms, Ws[1], b8s[1], out0)
    p2 = _agg_kernel(cols[4], cols[5], xs2)
    out2, diff2 = _matmul_step(2, p2, norms, Ws[2], b8s[2], out1)

    o0, o1, o2 = out0[:N], out1[:N], out2[:N]
    return (o0, o1, o2, o0, diff1[:N], diff2[:N])


# speedup vs baseline: 3.1084x; 1.0084x over previous
import functools

import jax
import jax.numpy as jnp
from jax import lax
from jax.experimental import pallas as pl
from jax.experimental.pallas import tpu as pltpu
from jax.experimental.pallas import tpu_sc as plsc

N = 10000
D = 128
E = 320000
NPAD = 10240
NC = 2
NS = 16
NW = NC * NS
CH = 64
EPT = NPAD
EPAD = EPT * NW
NCHUNK = EPT // CH
NB = 4
NG = NCHUNK // NB
RPT = NPAD // NS
RB = 1024

_mesh = plsc.VectorSubcoreMesh(core_axis_name="c", subcore_axis_name="s")


NROW = NPAD // 128


@functools.partial(
    pl.kernel,
    out_type=jax.ShapeDtypeStruct((NW, 6, NROW, 128), jnp.float32),
    mesh=_mesh,
    scratch_types=[
        pltpu.VMEM((NCHUNK // 10, CH), jnp.int32),
        pltpu.VMEM((NPAD,), jnp.float32),
        pltpu.VMEM((NROW, 128), jnp.float32),
        pltpu.VMEM((1, NROW), jnp.int32),
        pltpu.SemaphoreType.DMA,
    ],
    compiler_params=pltpu.CompilerParams(needs_layout_passes=False),
)
def _deg_kernel(i0, i1, i2, i3, i4, i5, out, idx_v, hist_v, hist2_v,
                ii_v, sem):
    c = lax.axis_index("c")
    s = lax.axis_index("s")
    w = c * NS + s

    def _fill_ii(m, carry):
        ii_v[0, pl.ds(m * 16, 16)] = lax.iota(jnp.int32, 16) + m * 16
        return carry
    lax.fori_loop(0, NROW // 16, _fill_ii, 0)

    ones = jnp.ones((16,), jnp.float32)
    NSEC = 10
    SEC = NCHUNK // NSEC
    for h, href in enumerate((i0, i1, i2, i3, i4, i5)):
        def _zero(i, carry):
            hist_v[pl.ds(i * 16, 16)] = jnp.zeros((16,), jnp.float32)
            return carry
        lax.fori_loop(0, NPAD // 16, _zero, 0)

        for sec in range(NSEC):
            pltpu.sync_copy(href.at[w, pl.ds(sec * SEC, SEC)], idx_v)

            def _acc(j, carry):
                for k in range(CH // 16):
                    v = idx_v[j, pl.ds(k * 16, 16)]
                    plsc.addupdate_scatter(hist_v, [v], ones)
                return carry
            lax.fori_loop(0, SEC, _acc, 0)

        def _pack(i, carry):
            for k in range(128 // 16):
                hist2_v[i, pl.ds(k * 16, 16)] = (
                    hist_v[pl.ds(i * 128 + k * 16, 16)])
            return carry
        lax.fori_loop(0, NROW, _pack, 0)
        pltpu.async_copy(hist2_v, out.at[w, h].at[ii_v.at[0]], sem).wait()


@functools.partial(
    pl.kernel,
    out_type=jax.ShapeDtypeStruct((NC, NPAD, D), jnp.float32),
    mesh=_mesh,
    scratch_types=(
        [pltpu.VMEM((2, NB, CH), jnp.int32),
         pltpu.VMEM((2, NB, CH), jnp.int32),
         pltpu.VMEM((NB, CH, D), jnp.float32),
         pltpu.VMEM((RPT // CH, CH), jnp.int32),
         pltpu.VMEM_SHARED((NPAD, D), jnp.float32)]
        + [pltpu.SemaphoreType.DMA] * (2 * NB + 2)
    ),
)
def _agg_kernel(srcC, dstC, xs, out, si_v, di_v, rows_v, ii_v, agg_sh,
                *sems):
    c = lax.axis_index("c")
    s = lax.axis_index("s")
    w = c * NS + s
    gs = sems[:NB]
    ss = sems[NB:2 * NB]
    isem = sems[2 * NB:]
    NZ = RPT // CH

    def _z(i, carry):
        for k in range(D // 16):
            rows_v[0, i, pl.ds(k * 16, 16)] = jnp.zeros((16,), jnp.float32)
        return carry
    lax.fori_loop(0, CH, _z, 0)
    for r in range(NZ):
        def _fill_ii(m, carry, r=r):
            ii_v[r, pl.ds(m * 16, 16)] = (lax.iota(jnp.int32, 16)
                                          + s * RPT + r * CH + m * 16)
            return carry
        lax.fori_loop(0, CH // 16, _fill_ii, 0)
    for r in range(NZ):
        pltpu.async_copy(rows_v.at[0], agg_sh.at[ii_v.at[r]], isem[0])
    for r in range(NZ):
        pltpu.make_async_copy(rows_v.at[0], agg_sh.at[ii_v.at[0]],
                              isem[0]).wait()
    plsc.subcore_barrier()

    def load_idx(g, p):
        pltpu.async_copy(srcC.at[w, pl.ds(g * NB, NB)], si_v.at[p], isem[p])
        pltpu.async_copy(dstC.at[w, pl.ds(g * NB, NB)], di_v.at[p], isem[p])

    def wait_idx(p):
        pltpu.make_async_copy(srcC.at[0, pl.ds(0, NB)], si_v.at[p],
                              isem[p]).wait()
        pltpu.make_async_copy(dstC.at[0, pl.ds(0, NB)], di_v.at[p],
                              isem[p]).wait()

    def fire_g(p, b):
        pltpu.async_copy(xs.at[si_v.at[p, b]], rows_v.at[b], gs[b])

    def wait_g(b):
        pltpu.make_async_copy(xs.at[si_v.at[0, 0]], rows_v.at[b],
                              gs[b]).wait()

    def fire_s(p, b):
        pltpu.async_copy(rows_v.at[b], agg_sh.at[di_v.at[p, b]], ss[b],
                         add=True)

    def wait_s(b):
        pltpu.make_async_copy(rows_v.at[b], agg_sh.at[di_v.at[0, 0]],
                              ss[b]).wait()

    load_idx(0, 0)
    load_idx(1, 1)
    wait_idx(0)
    for b in range(NB):
        fire_g(0, b)

    def _sg(gg, carry):
        for p in range(2):
            g = 2 * gg + p
            for b in range(NB):
                wait_g(b)
                fire_s(p, b)

            @pl.when(g < NG - 1)
            def _wi(p=p):
                wait_idx(1 - p)
            for b in range(NB):
                @pl.when(g < NG - 1)
                def _nx(p=p, b=b):
                    wait_s(b)
                    fire_g(1 - p, b)

            @pl.when(g + 2 < NG)
            def _ld(g=g, p=p):
                load_idx(g + 2, p)
        return carry

    lax.fori_loop(0, NG // 2, _sg, 0)
    for b in range(NB):
        wait_s(b)
    plsc.subcore_barrier()

    def fire_o(r, b):
        pltpu.async_copy(agg_sh.at[ii_v.at[r]], rows_v.at[b], gs[b])

    def wait_o(b):
        pltpu.make_async_copy(agg_sh.at[ii_v.at[0]], rows_v.at[b],
                              gs[b]).wait()

    def fire_w(r, b):
        pltpu.async_copy(rows_v.at[b], out.at[c].at[ii_v.at[r]], ss[b])

    def wait_w(b):
        pltpu.make_async_copy(rows_v.at[b], out.at[c].at[ii_v.at[0]],
                              ss[b]).wait()

    fire_o(0, 0)
    for r in range(NZ):
        b = r % 2
        wait_o(b)
        if r >= 1:
            wait_w(1 - b)
        if r + 1 < NZ:
            fire_o(r + 1, 1 - b)
        fire_w(r, b)
    wait_w((NZ - 1) % 2)


def _norms_xs0_body(dref, f0_ref, norms_ref, xs0_ref):
    d = jnp.sum(dref[...], axis=1)
    n6 = lax.rsqrt(jnp.maximum(d, 1.0))
    y8 = jnp.concatenate(
        [n6, jnp.zeros((2, n6.shape[1]), jnp.float32)], axis=0)
    i8 = jnp.eye(8, dtype=jnp.float32)
    nt = lax.dot_general(y8, i8, (((0,), (0,)), ((), ())),
                         preferred_element_type=jnp.float32)
    norms_ref[...] = nt
    xs0_ref[...] = f0_ref[...] * nt[:, 0:1]


def _norms_xs0(deg, f0p):
    return pl.pallas_call(
        _norms_xs0_body,
        grid=(NPAD // RB,),
        in_specs=[
            pl.BlockSpec((6, NW, RB), lambda i: (0, 0, i)),
            pl.BlockSpec((RB, D), lambda i: (i, 0)),
        ],
        out_specs=[
            pl.BlockSpec((RB, 8), lambda i: (i, 0)),
            pl.BlockSpec((RB, D), lambda i: (i, 0)),
        ],
        out_shape=[
            jax.ShapeDtypeStruct((NPAD, 8), jnp.float32),
            jax.ShapeDtypeStruct((NPAD, D), jnp.float32),
        ],
    )(deg, f0p)


def _make_m_body(t):
    def body(*refs):
        if t == 0:
            pref, nref, wref, bref, oref, xref = refs
        elif t == 1:
            pref, nref, wref, bref, vref, oref, xref, dref = refs
        else:
            pref, nref, wref, bref, vref, oref, dref = refs
        nt = nref[...]
        gblk = (pref[0] + pref[1]) * nt[:, 2 * t + 1:2 * t + 2]
        o = jnp.dot(gblk, wref[...],
                    preferred_element_type=jnp.float32) + bref[0:1, :]
        oref[...] = o
        if t < 2:
            xref[...] = o * nt[:, 2 * t + 2:2 * t + 3]
        if t > 0:
            dref[...] = o - vref[...]
    return body


def _matmul_step(t, part, norms, W, b8, prev):
    in_specs = [
        pl.BlockSpec((2, RB, D), lambda i: (0, i, 0)),
        pl.BlockSpec((RB, 8), lambda i: (i, 0)),
        pl.BlockSpec((D, D), lambda i: (0, 0)),
        pl.BlockSpec((8, D), lambda i: (0, 0)),
    ]
    args = [part, norms, W, b8]
    if t > 0:
        in_specs.append(pl.BlockSpec((RB, D), lambda i: (i, 0)))
        args.append(prev)
    n_out = 2 if t != 1 else 3
    out_specs = [pl.BlockSpec((RB, D), lambda i: (i, 0))] * n_out
    out_shape = [jax.ShapeDtypeStruct((NPAD, D), jnp.float32)] * n_out
    return pl.pallas_call(
        _make_m_body(t),
        grid=(NPAD // RB,),
        in_specs=in_specs,
        out_specs=out_specs,
        out_shape=out_shape,
    )(*args)


def kernel(features_0, features_1, features_2, edge_index_0, edge_index_1,
           edge_index_2, W_0, W_1, W_2, b_0, b_1, b_2):
    del features_1, features_2

    cols = []
    for e in (edge_index_0, edge_index_1, edge_index_2):
        for r in range(2):
            cols.append(
                jnp.full((EPAD,), N, jnp.int32).at[:E].set(e[r])
                .reshape(NW, NCHUNK, CH))

    f0p = jnp.zeros((NPAD, D), jnp.float32).at[:N].set(features_0)
    b8s = [jnp.broadcast_to(b.reshape(1, D), (8, D)) for b in (b_0, b_1, b_2)]
    Ws = (W_0, W_1, W_2)

    degw = _deg_kernel(*cols)
    deg = degw.reshape(NW, 6, NPAD).transpose(1, 0, 2)
    norms, xs0 = _norms_xs0(deg, f0p)

    p0 = _agg_kernel(cols[0], cols[1], xs0)
    out0, xs1 = _matmul_step(0, p0, norms, Ws[0], b8s[0], None)
    p1 = _agg_kernel(cols[2], cols[3], xs1)
    out1, xs2, diff1 = _matmul_step(1, p1, norms, Ws[1], b8s[1], out0)
    p2 = _agg_kernel(cols[4], cols[5], xs2)
    out2, diff2 = _matmul_step(2, p2, norms, Ws[2], b8s[2], out1)

    o0, o1, o2 = out0[:N], out1[:N], out2[:N]
    return (o0, o1, o2, o0, diff1[:N], diff2[:N])


# speedup vs baseline: 8.4944x; 2.7327x over previous
import functools

import jax
import jax.numpy as jnp
from jax import lax
from jax.experimental import pallas as pl
from jax.experimental.pallas import tpu as pltpu
from jax.experimental.pallas import tpu_sc as plsc

N = 10000
D = 128
E = 320000
NPAD = 10240
NC = 2
NS = 16
NW = NC * NS
CH = 64
EPT = NPAD
EPAD = EPT * NW
NCHUNK = EPT // CH
NB = 4
NG = NCHUNK // NB
RPT = NPAD // NS
RB = 1024

_mesh = plsc.VectorSubcoreMesh(core_axis_name="c", subcore_axis_name="s")


NROW = NPAD // 128


@functools.partial(
    pl.kernel,
    out_type=jax.ShapeDtypeStruct((NW, 6, NROW, 128), jnp.float32),
    mesh=_mesh,
    scratch_types=[
        pltpu.VMEM((NCHUNK // 10, CH), jnp.int32),
        pltpu.VMEM((NPAD,), jnp.float32),
        pltpu.VMEM((NROW, 128), jnp.float32),
        pltpu.VMEM((1, NROW), jnp.int32),
        pltpu.SemaphoreType.DMA,
    ],
    compiler_params=pltpu.CompilerParams(needs_layout_passes=False),
)
def _deg_kernel(i0, i1, i2, i3, i4, i5, out, idx_v, hist_v, hist2_v,
                ii_v, sem):
    c = lax.axis_index("c")
    s = lax.axis_index("s")
    w = c * NS + s

    def _fill_ii(m, carry):
        ii_v[0, pl.ds(m * 16, 16)] = lax.iota(jnp.int32, 16) + m * 16
        return carry
    lax.fori_loop(0, NROW // 16, _fill_ii, 0)

    ones = jnp.ones((16,), jnp.float32)
    NSEC = 10
    SEC = NCHUNK // NSEC
    for h, href in enumerate((i0, i1, i2, i3, i4, i5)):
        def _zero(i, carry):
            hist_v[pl.ds(i * 16, 16)] = jnp.zeros((16,), jnp.float32)
            return carry
        lax.fori_loop(0, NPAD // 16, _zero, 0)

        for sec in range(NSEC):
            pltpu.sync_copy(href.at[w, pl.ds(sec * SEC, SEC)], idx_v)

            def _acc(j, carry):
                for k in range(CH // 16):
                    v = idx_v[j, pl.ds(k * 16, 16)]
                    plsc.addupdate_scatter(hist_v, [v], ones)
                return carry
            lax.fori_loop(0, SEC, _acc, 0)

        def _pack(i, carry):
            for k in range(128 // 16):
                hist2_v[i, pl.ds(k * 16, 16)] = (
                    hist_v[pl.ds(i * 128 + k * 16, 16)])
            return carry
        lax.fori_loop(0, NROW, _pack, 0)
        pltpu.async_copy(hist2_v, out.at[w, h].at[ii_v.at[0]], sem).wait()


@functools.partial(
    pl.kernel,
    out_type=jax.ShapeDtypeStruct((NC, NPAD, D), jnp.float32),
    mesh=_mesh,
    scratch_types=(
        [pltpu.VMEM((2, NB, CH), jnp.int32),
         pltpu.VMEM((2, NB, CH), jnp.int32),
         pltpu.VMEM((NB, CH, D), jnp.float32),
         pltpu.VMEM((RPT // CH, CH), jnp.int32),
         pltpu.VMEM_SHARED((NPAD, D), jnp.float32)]
        + [pltpu.SemaphoreType.DMA] * (2 * NB + 2)
    ),
)
def _agg_kernel(srcC, dstC, xs, out, si_v, di_v, rows_v, ii_v, agg_sh,
                *sems):
    c = lax.axis_index("c")
    s = lax.axis_index("s")
    w = c * NS + s
    gs = sems[:NB]
    ss = sems[NB:2 * NB]
    isem = sems[2 * NB:]
    NZ = RPT // CH

    def _z(i, carry):
        for k in range(D // 16):
            rows_v[0, i, pl.ds(k * 16, 16)] = jnp.zeros((16,), jnp.float32)
        return carry
    lax.fori_loop(0, CH, _z, 0)
    for r in range(NZ):
        def _fill_ii(m, carry, r=r):
            ii_v[r, pl.ds(m * 16, 16)] = (lax.iota(jnp.int32, 16)
                                          + s * RPT + r * CH + m * 16)
            return carry
        lax.fori_loop(0, CH // 16, _fill_ii, 0)
    for r in range(NZ):
        pltpu.async_copy(rows_v.at[0], agg_sh.at[ii_v.at[r]], isem[0])
    for r in range(NZ):
        pltpu.make_async_copy(rows_v.at[0], agg_sh.at[ii_v.at[0]],
                              isem[0]).wait()
    plsc.subcore_barrier()

    def load_idx(g, p):
        pltpu.async_copy(srcC.at[w, pl.ds(g * NB, NB)], si_v.at[p], isem[p])
        pltpu.async_copy(dstC.at[w, pl.ds(g * NB, NB)], di_v.at[p], isem[p])

    def wait_idx(p):
        pltpu.make_async_copy(srcC.at[0, pl.ds(0, NB)], si_v.at[p],
                              isem[p]).wait()
        pltpu.make_async_copy(dstC.at[0, pl.ds(0, NB)], di_v.at[p],
                              isem[p]).wait()

    def fire_g(p, b):
        pltpu.async_copy(xs.at[si_v.at[p, b]], rows_v.at[b], gs[b])

    def wait_g(b):
        pltpu.make_async_copy(xs.at[si_v.at[0, 0]], rows_v.at[b],
                              gs[b]).wait()

    def fire_s(p, b):
        pltpu.async_copy(rows_v.at[b], agg_sh.at[di_v.at[p, b]], ss[b],
                         add=True)

    def wait_s(b):
        pltpu.make_async_copy(rows_v.at[b], agg_sh.at[di_v.at[0, 0]],
                              ss[b]).wait()

    load_idx(0, 0)
    load_idx(1, 1)
    wait_idx(0)
    for b in range(NB):
        fire_g(0, b)

    def _sg(gg, carry):
        for p in range(2):
            g = 2 * gg + p
            for b in range(NB):
                wait_g(b)
                fire_s(p, b)

            @pl.when(g < NG - 1)
            def _wi(p=p):
                wait_idx(1 - p)
            for b in range(NB):
                @pl.when(g < NG - 1)
                def _nx(p=p, b=b):
                    wait_s(b)
                    fire_g(1 - p, b)

            @pl.when(g + 2 < NG)
            def _ld(g=g, p=p):
                load_idx(g + 2, p)
        return carry

    lax.fori_loop(0, NG // 2, _sg, 0)
    for b in range(NB):
        wait_s(b)
    plsc.subcore_barrier()

    def fire_o(r, b):
        pltpu.async_copy(agg_sh.at[ii_v.at[r]], rows_v.at[b], gs[b])

    def wait_o(b):
        pltpu.make_async_copy(agg_sh.at[ii_v.at[0]], rows_v.at[b],
                              gs[b]).wait()

    def fire_w(r, b):
        pltpu.async_copy(rows_v.at[b], out.at[c].at[ii_v.at[r]], ss[b])

    def wait_w(b):
        pltpu.make_async_copy(rows_v.at[b], out.at[c].at[ii_v.at[0]],
                              ss[b]).wait()

    fire_o(0, 0)
    for r in range(NZ):
        b = r % 2
        wait_o(b)
        if r >= 1:
            wait_w(1 - b)
        if r + 1 < NZ:
            fire_o(r + 1, 1 - b)
        fire_w(r, b)
    wait_w((NZ - 1) % 2)


def _norms_xs0_body(dref, f0_ref, norms_ref, xs0_ref):
    d = jnp.sum(dref[...], axis=1)
    n6 = lax.rsqrt(jnp.maximum(d, 1.0))
    y8 = jnp.concatenate(
        [n6, jnp.zeros((2, n6.shape[1]), jnp.float32)], axis=0)
    i8 = jnp.eye(8, dtype=jnp.float32)
    nt = lax.dot_general(y8, i8, (((0,), (0,)), ((), ())),
                         preferred_element_type=jnp.float32)
    norms_ref[...] = nt
    xs0_ref[...] = f0_ref[...] * nt[:, 0:1]


def _norms_xs0(deg, f0p):
    return pl.pallas_call(
        _norms_xs0_body,
        grid=(NPAD // RB,),
        in_specs=[
            pl.BlockSpec((6, NW, RB), lambda i: (0, 0, i)),
            pl.BlockSpec((RB, D), lambda i: (i, 0)),
        ],
        out_specs=[
            pl.BlockSpec((RB, 8), lambda i: (i, 0)),
            pl.BlockSpec((RB, D), lambda i: (i, 0)),
        ],
        out_shape=[
            jax.ShapeDtypeStruct((NPAD, 8), jnp.float32),
            jax.ShapeDtypeStruct((NPAD, D), jnp.float32),
        ],
    )(deg, f0p)


def _make_m_body(t):
    def body(*refs):
        if t == 0:
            pref, nref, wref, bref, oref, xref = refs
        elif t == 1:
            pref, nref, wref, bref, vref, oref, xref, dref = refs
        else:
            pref, nref, wref, bref, vref, oref, dref = refs
        nt = nref[...]
        gblk = (pref[0] + pref[1]) * nt[:, 2 * t + 1:2 * t + 2]
        o = jnp.dot(gblk, wref[...],
                    preferred_element_type=jnp.float32) + bref[0:1, :]
        oref[...] = o
        if t < 2:
            xref[...] = o * nt[:, 2 * t + 2:2 * t + 3]
        if t > 0:
            dref[...] = o - vref[...]
    return body


def _matmul_step(t, part, norms, W, b8, prev):
    in_specs = [
        pl.BlockSpec((2, RB, D), lambda i: (0, i, 0)),
        pl.BlockSpec((RB, 8), lambda i: (i, 0)),
        pl.BlockSpec((D, D), lambda i: (0, 0)),
        pl.BlockSpec((8, D), lambda i: (0, 0)),
    ]
    args = [part, norms, W, b8]
    if t > 0:
        in_specs.append(pl.BlockSpec((RB, D), lambda i: (i, 0)))
        args.append(prev)
    n_out = 2 if t != 1 else 3
    out_specs = [pl.BlockSpec((RB, D), lambda i: (i, 0))] * n_out
    out_shape = [jax.ShapeDtypeStruct((NPAD, D), jnp.float32)] * n_out
    return pl.pallas_call(
        _make_m_body(t),
        grid=(NPAD // RB,),
        in_specs=in_specs,
        out_specs=out_specs,
        out_shape=out_shape,
    )(*args)


def kernel(features_0, features_1, features_2, edge_index_0, edge_index_1,
           edge_index_2, W_0, W_1, W_2, b_0, b_1, b_2):
    del features_1, features_2

    pad_vals = N + (jnp.arange(EPAD - E, dtype=jnp.int32) % (NPAD - N))
    cols = []
    for e in (edge_index_0, edge_index_1, edge_index_2):
        for r in range(2):
            cols.append(
                jnp.full((EPAD,), N, jnp.int32).at[E:].set(pad_vals)
                .at[:E].set(e[r]).reshape(NW, NCHUNK, CH))

    f0p = jnp.zeros((NPAD, D), jnp.float32).at[:N].set(features_0)
    b8s = [jnp.broadcast_to(b.reshape(1, D), (8, D)) for b in (b_0, b_1, b_2)]
    Ws = (W_0, W_1, W_2)

    degw = _deg_kernel(*cols)
    deg = degw.reshape(NW, 6, NPAD).transpose(1, 0, 2)
    norms, xs0 = _norms_xs0(deg, f0p)

    p0 = _agg_kernel(cols[0], cols[1], xs0)
    out0, xs1 = _matmul_step(0, p0, norms, Ws[0], b8s[0], None)
    p1 = _agg_kernel(cols[2], cols[3], xs1)
    out1, xs2, diff1 = _matmul_step(1, p1, norms, Ws[1], b8s[1], out0)
    p2 = _agg_kernel(cols[4], cols[5], xs2)
    out2, diff2 = _matmul_step(2, p2, norms, Ws[2], b8s[2], out1)

    o0, o1, o2 = out0[:N], out1[:N], out2[:N]
    return (o0, o1, o2, o0, diff1[:N], diff2[:N])
